# baseline (speedup 1.0000x reference)
import functools

import jax
import jax.numpy as jnp
from jax import lax
from jax.experimental import pallas as pl
from jax.experimental.pallas import tpu as pltpu
from jax.experimental.pallas import tpu_sc as plsc

H = 32
V = 1537
VP = 1544
D = 5
F = 3
K = D * F
B, N = 16, 64
P = N * N
C = 128
HH = H // 2
TW = D * VP
PW = P // 2


def _fold_body(we_ref, w_ref, out_ref):
    out_ref[0] = jnp.dot(we_ref[...], w_ref[0], preferred_element_type=jnp.float32)


def _fold_tables(we_pad, w):
    return pl.pallas_call(
        _fold_body,
        grid=(D,),
        in_specs=[
            pl.BlockSpec((VP, H), lambda d: (0, 0)),
            pl.BlockSpec((1, H, H), lambda d: (d, 0, 0)),
        ],
        out_specs=pl.BlockSpec((1, VP, H), lambda d: (d, 0, 0)),
        out_shape=jax.ShapeDtypeStruct((D, VP, H), jnp.float32),
    )(we_pad, w)


_mesh = plsc.VectorSubcoreMesh(core_axis_name="c", subcore_axis_name="s")


@functools.partial(
    pl.kernel,
    out_type=jax.ShapeDtypeStruct((B, H, P), jnp.float32),
    mesh=_mesh,
    scratch_types=[
        pltpu.VMEM((TW * HH,), jnp.int32),
        pltpu.VMEM((C * K,), jnp.int32),
        pltpu.VMEM((C,), jnp.int32),
        pltpu.VMEM((H, C), jnp.float32),
    ],
    compiler_params=pltpu.CompilerParams(needs_layout_passes=False),
)
def _sc_kernel(tab_hbm, idx_hbm, sp_hbm, out_hbm, tab_v, idx_v, sp_v, out_v):
    cid = lax.axis_index("c")
    sid = lax.axis_index("s")
    wid = sid * 2 + cid
    b = wid % B
    ph = wid // B
    pltpu.sync_copy(tab_hbm, tab_v)
    iota = lax.iota(jnp.int32, 16)
    iota_k = iota * K

    def chunk_body(chunk, _):
        base = ph * PW + chunk * C
        pltpu.sync_copy(idx_hbm.at[b, pl.ds(base * K, C * K)], idx_v)
        pltpu.sync_copy(sp_hbm.at[b, pl.ds(base, C)], sp_v)

        def group_body(g, _):
            pos = g * 16 + iota
            vks = [
                (plsc.load_gather(idx_v, [g * (16 * K) + k + iota_k])
                 + (k // 3) * VP) * HH
                for k in range(K)
            ]
            s = plsc.load_gather(sp_v, [pos])
            s = jnp.where(s == 0, 1, s)
            s = jnp.where(s > 1, s - 1, s)
            s = jnp.minimum(jnp.maximum(s, 0), D)
            scale = 1.0 / (3.0 * s.astype(jnp.float32))
            for hp in range(HH):
                los, his = [], []
                for k in range(K):
                    word = plsc.load_gather(tab_v, [vks[k] + hp])
                    lo, hi = plsc.unpack(
                        plsc.bitcast(word, jnp.bfloat16),
                        format=plsc.PackFormat.INTERLEAVED)
                    los.append(lo)
                    his.append(hi)
                for vals in (los, his):
                    while len(vals) > 1:
                        vals[:] = [vals[i] + vals[i + 1]
                                   for i in range(0, len(vals) - 1, 2)] + (
                            [vals[-1]] if len(vals) % 2 else [])
                out_v[hp, pl.ds(g * 16, 16)] = los[0] * scale
                out_v[hp + HH, pl.ds(g * 16, 16)] = his[0] * scale
            return 0

        lax.fori_loop(0, C // 16, group_body, 0)
        pltpu.sync_copy(out_v, out_hbm.at[b, pl.ds(0, H), pl.ds(base, C)])
        return 0

    lax.fori_loop(0, PW // C, chunk_body, 0)


def kernel(attn_bias, spatial_pos, x, edge_input, attn_edge_type,
           edge_encoder_weight, edge_dis_encoder_weight):
    we_pad = jnp.pad(edge_encoder_weight, ((0, VP - V), (0, 0)))
    w = edge_dis_encoder_weight.reshape(-1, H, H)[:D]
    tc = _fold_tables(we_pad, w)
    perm = [c for i in range(HH) for c in (i, i + HH)]
    tab_bf = tc.reshape(TW, H)[:, perm].astype(jnp.bfloat16)
    tab = lax.bitcast_convert_type(
        tab_bf.reshape(TW, HH, 2), jnp.int32).reshape(TW * HH)
    idx = edge_input.reshape(B, P * K).astype(jnp.int32)
    sp = spatial_pos.reshape(B, P).astype(jnp.int32)
    out = _sc_kernel(tab, idx, sp)
    return out.reshape(B, H, N, N)


# speedup vs baseline: 1.3716x; 1.3716x over previous
import functools

import jax
import jax.numpy as jnp
from jax import lax
from jax.experimental import pallas as pl
from jax.experimental.pallas import tpu as pltpu
from jax.experimental.pallas import tpu_sc as plsc

H = 32
V = 1537
VP = 1544
D = 5
F = 3
K = D * F
B, N = 16, 64
P = N * N
C = 128
HH = H // 2
TW = D * VP


def _fold_body(we_ref, w_ref, out_ref):
    out_ref[0] = jnp.dot(we_ref[...], w_ref[0], preferred_element_type=jnp.float32)


def _fold_tables(we_pad, w):
    return pl.pallas_call(
        _fold_body,
        grid=(D,),
        in_specs=[
            pl.BlockSpec((VP, H), lambda d: (0, 0)),
            pl.BlockSpec((1, H, H), lambda d: (d, 0, 0)),
        ],
        out_specs=pl.BlockSpec((1, VP, H), lambda d: (d, 0, 0)),
        out_shape=jax.ShapeDtypeStruct((D, VP, H), jnp.float32),
    )(we_pad, w)


_mesh = plsc.VectorSubcoreMesh(core_axis_name="c", subcore_axis_name="s")


@functools.partial(
    pl.kernel,
    out_type=jax.ShapeDtypeStruct((B, H, P), jnp.float32),
    mesh=_mesh,
    scratch_types=[
        pltpu.VMEM((HH * TW,), jnp.float32),
        pltpu.VMEM((C * K,), jnp.int32),
        pltpu.VMEM((C,), jnp.int32),
        pltpu.VMEM((HH, C), jnp.float32),
    ],
    compiler_params=pltpu.CompilerParams(needs_layout_passes=False),
)
def _sc_kernel(tab_hbm, idx_hbm, sp_hbm, out_hbm, tab_v, idx_v, sp_v, out_v):
    cid = lax.axis_index("c")
    sid = lax.axis_index("s")
    wid = sid * 2 + cid
    b = wid % B
    half = wid // B
    pltpu.sync_copy(tab_hbm.at[pl.ds(half * (HH * TW), HH * TW)], tab_v)
    iota = lax.iota(jnp.int32, 16)
    iota_k = iota * K

    def chunk_body(chunk, _):
        base = chunk * C
        pltpu.sync_copy(idx_hbm.at[b, pl.ds(base * K, C * K)], idx_v)
        pltpu.sync_copy(sp_hbm.at[b, pl.ds(base, C)], sp_v)

        @plsc.parallel_loop(0, C // 16, unroll=2)
        def group_body(g):
            pos = g * 16 + iota
            vks = [
                plsc.load_gather(idx_v, [g * (16 * K) + k + iota_k])
                + (k // 3) * VP
                for k in range(K)
            ]
            s = plsc.load_gather(sp_v, [pos])
            s = jnp.where(s == 0, 1, s)
            s = jnp.where(s > 1, s - 1, s)
            s = jnp.minimum(jnp.maximum(s, 0), D)
            scale = 1.0 / (3.0 * s.astype(jnp.float32))
            for h in range(HH):
                vals = [plsc.load_gather(tab_v, [vks[k] + h * TW]) for k in range(K)]
                while len(vals) > 1:
                    vals = [vals[i] + vals[i + 1] for i in range(0, len(vals) - 1, 2)] + (
                        [vals[-1]] if len(vals) % 2 else [])
                out_v[h, pl.ds(g * 16, 16)] = vals[0] * scale

        pltpu.sync_copy(out_v, out_hbm.at[b, pl.ds(half * HH, HH), pl.ds(base, C)])
        return 0

    lax.fori_loop(0, P // C, chunk_body, 0)


def kernel(attn_bias, spatial_pos, x, edge_input, attn_edge_type,
           edge_encoder_weight, edge_dis_encoder_weight):
    we_pad = jnp.pad(edge_encoder_weight, ((0, VP - V), (0, 0)))
    w = edge_dis_encoder_weight.reshape(-1, H, H)[:D]
    tc = _fold_tables(we_pad, w)
    tab = tc.transpose(2, 0, 1).reshape(H * TW)
    idx = edge_input.reshape(B, P * K).astype(jnp.int32)
    sp = spatial_pos.reshape(B, P).astype(jnp.int32)
    out = _sc_kernel(tab, idx, sp)
    return out.reshape(B, H, N, N)
